# Initial kernel scaffold; baseline (speedup 1.0000x reference)
import jax
import jax.numpy as jnp
from jax.experimental import pallas as pl


def kernel(x, adj_vals, edge_index, W1, b1, W2, b2):
    raise NotImplementedError("write your pallas kernel here")


# speedup vs baseline: 3.5788x; 3.5788x over previous
import functools

import jax
import jax.numpy as jnp
from jax import lax
from jax.experimental import pallas as pl
from jax.experimental.pallas import tpu as pltpu
from jax.experimental.pallas import tpu_sc as plsc

N = 10000
E = 160000
D = 256
HID = 512
HALF = D // 2
L = 16
NC = 2
NS = 16
B = 128
EPT = E // NS
C = -(-EPT // B)
PT = C * B
ROWS_PT = N // NS
FR = 125
NBLK = ROWS_PT // FR


def _spmm_body(table, srcs, dsts, vals, bias, out,
               idx_v, dst_v, vals_v, rows_v, bias_v, fill_v, acc, sem):
    c = lax.axis_index("c")
    s = lax.axis_index("s")

    pltpu.sync_copy(bias.at[c], bias_v)

    @pl.loop(0, FR)
    def _fill(r):
        for j in range(HALF // L):
            sl = pl.ds(j * L, L)
            fill_v[r, sl] = bias_v[sl]

    base = s * ROWS_PT
    for kk in range(NBLK):
        pltpu.sync_copy(fill_v, acc.at[pl.ds(base + kk * FR, FR)])
    plsc.subcore_barrier()

    @pl.loop(0, C)
    def _chunk(k):
        pltpu.sync_copy(srcs.at[s, k], idx_v)
        pltpu.sync_copy(dsts.at[s, k], dst_v)
        pltpu.sync_copy(vals.at[s, k], vals_v)
        for j in range(B // L):
            sl = pl.ds(j * L, L)
            idx_v[sl] = idx_v[sl] + c
        pltpu.async_copy(table.at[idx_v], rows_v, sem).wait()

        @pl.loop(0, B // L)
        def _scale(gi):
            vv = vals_v[pl.ds(gi * L, L)]
            for i in range(L):
                r = gi * L + i
                v = vv[i]
                for j in range(HALF // L):
                    sl = pl.ds(j * L, L)
                    rows_v[r, sl] = rows_v[r, sl] * v

        pltpu.sync_copy(rows_v, acc.at[dst_v], add=True)

    plsc.subcore_barrier()

    for kk in range(NBLK):
        sl = pl.ds(base + kk * FR, FR)
        pltpu.sync_copy(acc.at[sl], out.at[sl, c])


@functools.partial(jax.jit, donate_argnums=())
def _spmm(table, srcs, dsts, vals, bias):
    mesh = plsc.VectorSubcoreMesh(core_axis_name="c", subcore_axis_name="s")
    return pl.kernel(
        _spmm_body,
        out_type=jax.ShapeDtypeStruct((N, NC, HALF), jnp.float32),
        mesh=mesh,
        scratch_types=[
            pltpu.VMEM((B,), jnp.int32),
            pltpu.VMEM((B,), jnp.int32),
            pltpu.VMEM((B,), jnp.float32),
            pltpu.VMEM((B, HALF), jnp.float32),
            pltpu.VMEM((HALF,), jnp.float32),
            pltpu.VMEM((FR, HALF), jnp.float32),
            pltpu.VMEM_SHARED((N, HALF), jnp.float32),
            pltpu.SemaphoreType.DMA,
        ],
    )(table, srcs, dsts, vals, bias)


def _dense_body(s_ref, w1_ref, b1_ref, w2_ref, o_ref):
    a = s_ref[...]
    h = jnp.dot(a, w1_ref[...], preferred_element_type=jnp.float32)
    h = jnp.maximum(h + b1_ref[...], 0.0)
    o_ref[...] = jnp.dot(h, w2_ref[...], preferred_element_type=jnp.float32)


def _dense(s, W1, b1, W2):
    M = 1000
    return pl.pallas_call(
        _dense_body,
        grid=(N // M,),
        in_specs=[
            pl.BlockSpec((M, D), lambda i: (i, 0)),
            pl.BlockSpec((D, HID), lambda i: (0, 0)),
            pl.BlockSpec((1, HID), lambda i: (0, 0)),
            pl.BlockSpec((HID, D), lambda i: (0, 0)),
        ],
        out_specs=pl.BlockSpec((M, D), lambda i: (i, 0)),
        out_shape=jax.ShapeDtypeStruct((N, D), jnp.float32),
    )(s, W1, b1.reshape(1, HID), W2)


def kernel(x, adj_vals, edge_index, W1, b1, W2, b2):
    src = edge_index[0].astype(jnp.int32)
    dst = edge_index[1].astype(jnp.int32)
    pad = NS * PT - E
    srcs = jnp.pad(src * 2, (0, pad)).reshape(NS, C, B)
    dsts = jnp.pad(dst, (0, pad)).reshape(NS, C, B)
    vals = jnp.pad(adj_vals, (0, pad)).reshape(NS, C, B)

    zero_bias = jnp.zeros((NC, HALF), jnp.float32)
    s3 = _spmm(x.reshape(N * NC, HALF), srcs, dsts, vals, zero_bias)
    g = _dense(s3.reshape(N, D), W1, b1, W2)
    out3 = _spmm(g.reshape(N * NC, HALF), srcs, dsts, vals,
                 b2.reshape(NC, HALF))
    return out3.reshape(N, D)


# speedup vs baseline: 7.6247x; 2.1305x over previous
import jax
import jax.numpy as jnp
from jax import lax
from jax.experimental import pallas as pl
from jax.experimental.pallas import tpu as pltpu
from jax.experimental.pallas import tpu_sc as plsc

N = 10000
E = 160000
D = 256
HID = 512
HALF = D // 2
L = 16
NC = 2
NS = 16
EPT = E // NS
B = 80
C = EPT // B
ROWS_PT = N // NS
IB = 80
IB_TAIL = ROWS_PT - (ROWS_PT // IB) * IB


def _spmm_body(table, srcs, dsts, vals, bias, out,
               idx_v, valsb, dstb, rows0, rows1, bias_v, acc,
               sg0, sg1, ss0, ss1, sd0, sd1, sv0, sv1):
    c = lax.axis_index("c")
    s = lax.axis_index("s")
    base = s * ROWS_PT

    pltpu.sync_copy(srcs.at[s], idx_v)

    @pl.loop(0, C)
    def _xform(r):
        for j in range(B // L):
            sl = pl.ds(j * L, L)
            idx_v[r, sl] = idx_v[r, sl] * 2 + c

    pltpu.sync_copy(bias.at[c], bias_v)

    @pl.loop(0, IB)
    def _fill(r):
        for j in range(HALF // L):
            sl = pl.ds(j * L, L)
            rows0[r, sl] = bias_v[sl]

    for kk in range(ROWS_PT // IB):
        pltpu.sync_copy(rows0, acc.at[pl.ds(base + kk * IB, IB)])
    if IB_TAIL:
        pltpu.sync_copy(rows0.at[pl.ds(0, IB_TAIL)],
                        acc.at[pl.ds(base + (ROWS_PT // IB) * IB, IB_TAIL)])

    plsc.subcore_barrier()

    def scale(rows, b):
        @pl.loop(0, B // L)
        def _scale(gi):
            vv = valsb[b, pl.ds(gi * L, L)]
            for i in range(L):
                r = gi * L + i
                v = vv[i]
                for j in range(HALF // L):
                    sl = pl.ds(j * L, L)
                    rows[r, sl] = rows[r, sl] * v

    def gather_start(k, rows, sem):
        pltpu.async_copy(table.at[idx_v.at[k]], rows, sem)

    def gather_wait(rows, sem):
        pltpu.make_async_copy(table.at[idx_v.at[0]], rows, sem).wait()

    def dst_start(k, b, sem):
        pltpu.async_copy(dsts.at[s, k], dstb.at[b], sem)

    def dst_wait(b, sem):
        pltpu.make_async_copy(dsts.at[s, 0], dstb.at[b], sem).wait()

    def vals_start(k, b, sem):
        pltpu.async_copy(vals.at[s, k], valsb.at[b], sem)

    def vals_wait(b, sem):
        pltpu.make_async_copy(vals.at[s, 0], valsb.at[b], sem).wait()

    def scatter_start(rows, b, sem):
        pltpu.async_copy(rows, acc.at[dstb.at[b]], sem, add=True)

    def scatter_wait(rows, sem):
        pltpu.make_async_copy(rows, acc.at[dstb.at[0]], sem).wait()

    dst_start(0, 0, sd0)
    vals_start(0, 0, sv0)
    gather_start(0, rows0, sg0)

    @pl.loop(0, C // 2)
    def _pipe(i):
        k0 = 2 * i
        k1 = k0 + 1
        @pl.when(i > 0)
        def _():
            scatter_wait(rows1, ss1)

        gather_start(k1, rows1, sg1)
        dst_start(k1, 1, sd1)
        vals_start(k1, 1, sv1)
        gather_wait(rows0, sg0)
        vals_wait(0, sv0)
        scale(rows0, 0)
        dst_wait(0, sd0)
        scatter_start(rows0, 0, ss0)
        gather_wait(rows1, sg1)
        vals_wait(1, sv1)
        scale(rows1, 1)
        dst_wait(1, sd1)
        scatter_start(rows1, 1, ss1)

        @pl.when(k1 + 1 < C)
        def _():
            scatter_wait(rows0, ss0)
            gather_start(k1 + 1, rows0, sg0)
            dst_start(k1 + 1, 0, sd0)
            vals_start(k1 + 1, 0, sv0)

    if C % 2:
        gather_wait(rows0, sg0)
        vals_wait(0, sv0)
        scale(rows0, 0)
        dst_wait(0, sd0)
        scatter_start(rows0, 0, ss0)
        scatter_wait(rows0, ss0)
    scatter_wait(rows1, ss1)

    plsc.subcore_barrier()

    for kk in range(ROWS_PT // IB):
        sl = pl.ds(base + kk * IB, IB)
        pltpu.sync_copy(acc.at[sl], out.at[sl, c])
    if IB_TAIL:
        sl = pl.ds(base + (ROWS_PT // IB) * IB, IB_TAIL)
        pltpu.sync_copy(acc.at[sl], out.at[sl, c])


def _spmm(table, srcs, dsts, vals, bias):
    mesh = plsc.VectorSubcoreMesh(core_axis_name="c", subcore_axis_name="s")
    return pl.kernel(
        _spmm_body,
        out_type=jax.ShapeDtypeStruct((N, NC, HALF), jnp.float32),
        mesh=mesh,
        scratch_types=[
            pltpu.VMEM((C, B), jnp.int32),
            pltpu.VMEM((2, B), jnp.float32),
            pltpu.VMEM((2, B), jnp.int32),
            pltpu.VMEM((B, HALF), jnp.float32),
            pltpu.VMEM((B, HALF), jnp.float32),
            pltpu.VMEM((HALF,), jnp.float32),
            pltpu.VMEM_SHARED((N, HALF), jnp.float32),
            pltpu.SemaphoreType.DMA,
            pltpu.SemaphoreType.DMA,
            pltpu.SemaphoreType.DMA,
            pltpu.SemaphoreType.DMA,
            pltpu.SemaphoreType.DMA,
            pltpu.SemaphoreType.DMA,
            pltpu.SemaphoreType.DMA,
            pltpu.SemaphoreType.DMA,
        ],
    )(table, srcs, dsts, vals, bias)


def _dense_body(s_ref, w1_ref, b1_ref, w2_ref, o_ref):
    a = s_ref[...]
    h = jnp.dot(a, w1_ref[...], preferred_element_type=jnp.float32)
    h = jnp.maximum(h + b1_ref[...], 0.0)
    o_ref[...] = jnp.dot(h, w2_ref[...], preferred_element_type=jnp.float32)


def _dense(s, W1, b1, W2):
    M = 1000
    return pl.pallas_call(
        _dense_body,
        grid=(N // M,),
        in_specs=[
            pl.BlockSpec((M, D), lambda i: (i, 0)),
            pl.BlockSpec((D, HID), lambda i: (0, 0)),
            pl.BlockSpec((1, HID), lambda i: (0, 0)),
            pl.BlockSpec((HID, D), lambda i: (0, 0)),
        ],
        out_specs=pl.BlockSpec((M, D), lambda i: (i, 0)),
        out_shape=jax.ShapeDtypeStruct((N, D), jnp.float32),
    )(s, W1, b1.reshape(1, HID), W2)


def kernel(x, adj_vals, edge_index, W1, b1, W2, b2):
    src = edge_index[0].astype(jnp.int32)
    dst = edge_index[1].astype(jnp.int32)
    srcs = src.reshape(NS, C, B)
    dsts = dst.reshape(NS, C, B)
    vals = adj_vals.reshape(NS, C, B)

    zero_bias = jnp.zeros((NC, HALF), jnp.float32)
    s3 = _spmm(x.reshape(N * NC, HALF), srcs, dsts, vals, zero_bias)
    g = _dense(s3.reshape(N, D), W1, b1, W2)
    out3 = _spmm(g.reshape(N * NC, HALF), srcs, dsts, vals,
                 b2.reshape(NC, HALF))
    return out3.reshape(N, D)


# speedup vs baseline: 8.2910x; 1.0874x over previous
import jax
import jax.numpy as jnp
from jax import lax
from jax.experimental import pallas as pl
from jax.experimental.pallas import tpu as pltpu
from jax.experimental.pallas import tpu_sc as plsc

N = 10000
E = 160000
D = 256
HID = 512
HALF = D // 2
L = 16
NC = 2
NS = 16
EPT = E // NS
B = 80
C = EPT // B
R = 3
ROWS_PT = N // NS
IB = 80
IB_TAIL = ROWS_PT - (ROWS_PT // IB) * IB


def _make_spmm_body():
    def _spmm_body(table, srcs, dsts, vals, bias, out,
                   idxb, dstb, valsb, rows0, rows1, rows2, bias_v, acc,
                   sg0, sg1, sg2, ss0, ss1, ss2, si0, si1, si2,
                   sd0, sd1, sd2, sv0, sv1, sv2):
        c = lax.axis_index("c")
        s = lax.axis_index("s")
        base = s * ROWS_PT
        rows = (rows0, rows1, rows2)
        sg = (sg0, sg1, sg2)
        ss = (ss0, ss1, ss2)
        si = (si0, si1, si2)
        sd = (sd0, sd1, sd2)
        sv = (sv0, sv1, sv2)

        pltpu.sync_copy(bias.at[c], bias_v)

        @pl.loop(0, IB)
        def _fill(r):
            for j in range(HALF // L):
                sl = pl.ds(j * L, L)
                rows0[r, sl] = bias_v[sl]

        for kk in range(ROWS_PT // IB):
            pltpu.sync_copy(rows0, acc.at[pl.ds(base + kk * IB, IB)])
        if IB_TAIL:
            pltpu.sync_copy(
                rows0.at[pl.ds(0, IB_TAIL)],
                acc.at[pl.ds(base + (ROWS_PT // IB) * IB, IB_TAIL)])

        plsc.subcore_barrier()

        def prefetch(k, j):
            pltpu.async_copy(srcs.at[s, k], idxb.at[j], si[j])
            pltpu.async_copy(dsts.at[s, k], dstb.at[j], sd[j])
            pltpu.async_copy(vals.at[s, k], valsb.at[j], sv[j])

        def gather_issue(k, j):
            pltpu.make_async_copy(srcs.at[s, 0], idxb.at[j], si[j]).wait()
            for jj in range(B // L):
                sl = pl.ds(jj * L, L)
                idxb[j, sl] = idxb[j, sl] * 2 + c
            pltpu.async_copy(table.at[idxb.at[j]], rows[j], sg[j])

        def scatter_wait(j):
            pltpu.make_async_copy(rows[j], acc.at[dstb.at[j]], ss[j]).wait()

        def process(k, j):
            pltpu.make_async_copy(table.at[idxb.at[j]], rows[j],
                                  sg[j]).wait()
            pltpu.make_async_copy(vals.at[s, 0], valsb.at[j], sv[j]).wait()

            @pl.loop(0, B // L)
            def _scale(gi):
                vv = valsb[j, pl.ds(gi * L, L)]
                for i in range(L):
                    r = gi * L + i
                    v = vv[i]
                    for jj in range(HALF // L):
                        sl = pl.ds(jj * L, L)
                        rows[j][r, sl] = rows[j][r, sl] * v

            pltpu.make_async_copy(dsts.at[s, 0], dstb.at[j], sd[j]).wait()
            pltpu.async_copy(rows[j], acc.at[dstb.at[j]], ss[j], add=True)

        def emit_step(k, j, swait, pref, gissue):
            jm1 = (j + 2) % R
            jp1 = (j + 1) % R
            if swait:
                scatter_wait(jm1)
            if pref:
                prefetch(k + 2, jm1)
            if gissue:
                gather_issue(k + 1, jp1)
            process(k, j)

        prefetch(0, 0)
        prefetch(1, 1)
        gather_issue(0, 0)
        emit_step(0, 0, False, True, True)
        emit_step(1, 1, True, True, True)
        emit_step(2, 2, True, True, True)

        @pl.loop(1, (C - 3) // R + 1)
        def _steady(g):
            for jj in range(R):
                emit_step(R * g + jj, jj, True, True, True)

        emit_step(C - 2, (C - 2) % R, True, False, True)
        emit_step(C - 1, (C - 1) % R, True, False, False)
        scatter_wait((C - 1) % R)

        plsc.subcore_barrier()

        for kk in range(ROWS_PT // IB):
            sl = pl.ds(base + kk * IB, IB)
            pltpu.sync_copy(acc.at[sl], out.at[sl, c])
        if IB_TAIL:
            sl = pl.ds(base + (ROWS_PT // IB) * IB, IB_TAIL)
            pltpu.sync_copy(acc.at[sl], out.at[sl, c])

    return _spmm_body


def _spmm(table, srcs, dsts, vals, bias):
    mesh = plsc.VectorSubcoreMesh(core_axis_name="c", subcore_axis_name="s")
    out_shape = (N, NC, HALF)
    return pl.kernel(
        _make_spmm_body(),
        out_type=jax.ShapeDtypeStruct(out_shape, jnp.float32),
        mesh=mesh,
        scratch_types=[
            pltpu.VMEM((R, B), jnp.int32),
            pltpu.VMEM((R, B), jnp.int32),
            pltpu.VMEM((R, B), jnp.float32),
            pltpu.VMEM((B, HALF), jnp.float32),
            pltpu.VMEM((B, HALF), jnp.float32),
            pltpu.VMEM((B, HALF), jnp.float32),
            pltpu.VMEM((HALF,), jnp.float32),
            pltpu.VMEM_SHARED((N, HALF), jnp.float32),
        ] + [pltpu.SemaphoreType.DMA] * 15,
    )(table, srcs, dsts, vals, bias)


def _dense_body(s_ref, w1_ref, b1_ref, w2_ref, o_ref):
    a = s_ref[...]
    h = jnp.dot(a, w1_ref[...], preferred_element_type=jnp.float32)
    h = jnp.maximum(h + b1_ref[...], 0.0)
    o_ref[...] = jnp.dot(h, w2_ref[...], preferred_element_type=jnp.float32)


def _dense(s, W1, b1, W2):
    M = 1000
    return pl.pallas_call(
        _dense_body,
        grid=(N // M,),
        in_specs=[
            pl.BlockSpec((M, D), lambda i: (i, 0)),
            pl.BlockSpec((D, HID), lambda i: (0, 0)),
            pl.BlockSpec((1, HID), lambda i: (0, 0)),
            pl.BlockSpec((HID, D), lambda i: (0, 0)),
        ],
        out_specs=pl.BlockSpec((M, D), lambda i: (i, 0)),
        out_shape=jax.ShapeDtypeStruct((N, D), jnp.float32),
    )(s, W1, b1.reshape(1, HID), W2)


def kernel(x, adj_vals, edge_index, W1, b1, W2, b2):
    src = edge_index[0].astype(jnp.int32)
    dst = edge_index[1].astype(jnp.int32)
    srcs = src.reshape(NS, C, B)
    dsts = dst.reshape(NS, C, B)
    vals = adj_vals.reshape(NS, C, B)

    zero_bias = jnp.zeros((NC, HALF), jnp.float32)
    s3 = _spmm(x.reshape(N * NC, HALF), srcs, dsts, vals, zero_bias)
    g = _dense(s3.reshape(N, D), W1, b1, W2)
    out3 = _spmm(g.reshape(N * NC, HALF), srcs, dsts, vals,
                 b2.reshape(NC, HALF))
    return out3.reshape(N, D)


# speedup vs baseline: 8.8911x; 1.0724x over previous
import jax
import jax.numpy as jnp
from jax import lax
from jax.experimental import pallas as pl
from jax.experimental.pallas import tpu as pltpu
from jax.experimental.pallas import tpu_sc as plsc

N = 10000
E = 160000
D = 256
HID = 512
HALF = D // 2
L = 16
NC = 2
NS = 16
EPT = E // NS
B = 80
C = EPT // B
R = 3
ROWS_PT = N // NS
IB = 80
IB_TAIL = ROWS_PT - (ROWS_PT // IB) * IB


def _spmm_body(table, srcs, dsts, vals, bias, out,
               idxb, dstb, valsb, rows0, rows1, rows2, bias_v, acc,
               *sems):
    rows = (rows0, rows1, rows2)
    c = lax.axis_index("c")
    s = lax.axis_index("s")
    base = s * ROWS_PT
    sg = sems[0:3]
    ss = sems[3:6]
    si = sems[6:9]
    sd = sems[9:12]
    sv = sems[12:15]

    pltpu.sync_copy(bias.at[c], bias_v)

    @pl.loop(0, IB)
    def _fill(r):
        for j in range(HALF // L):
            sl = pl.ds(j * L, L)
            rows0[r, sl] = bias_v[sl]

    for kk in range(ROWS_PT // IB):
        pltpu.sync_copy(rows0, acc.at[pl.ds(base + kk * IB, IB)])
    if IB_TAIL:
        pltpu.sync_copy(
            rows0.at[pl.ds(0, IB_TAIL)],
            acc.at[pl.ds(base + (ROWS_PT // IB) * IB, IB_TAIL)])

    plsc.subcore_barrier()

    def prefetch(k, j):
        pltpu.async_copy(srcs.at[s, k], idxb.at[j], si[j])
        pltpu.async_copy(dsts.at[s, k], dstb.at[j], sd[j])
        pltpu.async_copy(vals.at[s, k], valsb.at[j], sv[j])

    def gather_issue(k, j):
        pltpu.make_async_copy(srcs.at[s, 0], idxb.at[j], si[j]).wait()
        for jj in range(B // L):
            sl = pl.ds(jj * L, L)
            idxb[j, sl] = idxb[j, sl] * 2 + c
        pltpu.async_copy(table.at[idxb.at[j]], rows[j], sg[j])

    def scatter_wait(j):
        pltpu.make_async_copy(rows[j], acc.at[dstb.at[j]], ss[j]).wait()

    def process(k, j):
        pltpu.make_async_copy(table.at[idxb.at[j]], rows[j], sg[j]).wait()
        pltpu.make_async_copy(vals.at[s, 0], valsb.at[j], sv[j]).wait()

        @pl.loop(0, B // L)
        def _scale(gi):
            vv = valsb[j, pl.ds(gi * L, L)]
            for i in range(L):
                r = gi * L + i
                v = vv[i]
                for jj in range(HALF // L):
                    sl = pl.ds(jj * L, L)
                    rows[j][r, sl] = rows[j][r, sl] * v

        pltpu.make_async_copy(dsts.at[s, 0], dstb.at[j], sd[j]).wait()
        pltpu.async_copy(rows[j], acc.at[dstb.at[j]], ss[j], add=True)

    def emit_step(k, m, swait, pref, gissue):
        if swait:
            scatter_wait((m + 2) % R)
        if pref:
            prefetch(k + 2, (m + 2) % R)
        if gissue:
            gather_issue(k + 1, (m + 1) % R)
        process(k, m)

    prefetch(0, 0)
    prefetch(1, 1)
    gather_issue(0, 0)
    for k in range(R):
        emit_step(k, k, k >= 1, True, True)

    NG = (C - R - 2) // R

    @pl.loop(1, NG + 1)
    def _steady(g):
        for jj in range(R):
            emit_step(R * g + jj, jj, True, True, True)

    for k in range(R * (NG + 1), C):
        emit_step(k, k % R, True, k + 2 < C, k + 1 < C)
    scatter_wait((C - 1) % R)

    plsc.subcore_barrier()

    for kk in range(ROWS_PT // IB):
        sl = pl.ds(base + kk * IB, IB)
        pltpu.sync_copy(acc.at[sl], out.at[sl, c])
    if IB_TAIL:
        sl = pl.ds(base + (ROWS_PT // IB) * IB, IB_TAIL)
        pltpu.sync_copy(acc.at[sl], out.at[sl, c])


def _spmm(table, srcs, dsts, vals, bias):
    mesh = plsc.VectorSubcoreMesh(core_axis_name="c", subcore_axis_name="s")
    return pl.kernel(
        _spmm_body,
        out_type=jax.ShapeDtypeStruct((N, NC, HALF), jnp.float32),
        mesh=mesh,
        scratch_types=[
            pltpu.VMEM((R, B), jnp.int32),
            pltpu.VMEM((R, B), jnp.int32),
            pltpu.VMEM((R, B), jnp.float32),
            pltpu.VMEM((B, HALF), jnp.float32),
            pltpu.VMEM((B, HALF), jnp.float32),
            pltpu.VMEM((B, HALF), jnp.float32),
            pltpu.VMEM((HALF,), jnp.float32),
            pltpu.VMEM_SHARED((N, HALF), jnp.float32),
        ] + [pltpu.SemaphoreType.DMA] * 15,
    )(table, srcs, dsts, vals, bias)


def _dense_body(s_ref, w1_ref, b1_ref, w2_ref, o_ref):
    a = s_ref[...].reshape(-1, D)
    h = jnp.dot(a, w1_ref[...], preferred_element_type=jnp.float32)
    h = jnp.maximum(h + b1_ref[...], 0.0)
    g = jnp.dot(h, w2_ref[...], preferred_element_type=jnp.float32)
    o_ref[...] = g.reshape(-1, NC, HALF)


def _dense(s3, W1, b1, W2):
    M = 1000
    return pl.pallas_call(
        _dense_body,
        grid=(N // M,),
        in_specs=[
            pl.BlockSpec((M, NC, HALF), lambda i: (i, 0, 0)),
            pl.BlockSpec((D, HID), lambda i: (0, 0)),
            pl.BlockSpec((1, HID), lambda i: (0, 0)),
            pl.BlockSpec((HID, D), lambda i: (0, 0)),
        ],
        out_specs=pl.BlockSpec((M, NC, HALF), lambda i: (i, 0, 0)),
        out_shape=jax.ShapeDtypeStruct((N, NC, HALF), jnp.float32),
    )(s3, W1, b1.reshape(1, HID), W2)


def kernel(x, adj_vals, edge_index, W1, b1, W2, b2):
    src = edge_index[0].astype(jnp.int32)
    dst = edge_index[1].astype(jnp.int32)
    srcs = src.reshape(NS, C, B)
    dsts = dst.reshape(NS, C, B)
    vals = adj_vals.reshape(NS, C, B)

    zero_bias = jnp.zeros((NC, HALF), jnp.float32)
    s3 = _spmm(x.reshape(N * NC, HALF), srcs, dsts, vals, zero_bias)
    g3 = _dense(s3, W1, b1, W2)
    out3 = _spmm(g3.reshape(N * NC, HALF), srcs, dsts, vals,
                 b2.reshape(NC, HALF))
    return out3.reshape(N, D)
